# baseline (speedup 1.0000x reference)
import functools

import jax
import jax.numpy as jnp
from jax import lax
from jax.experimental import pallas as pl
from jax.experimental.pallas import tpu as pltpu
from jax.experimental.pallas import tpu_sc as plsc

_M_BLK = 256
_CHUNK_BLOCKS = (20, 20)
_DMA_ROWS = 16


@functools.lru_cache(maxsize=None)
def _sc_info():
    info = plsc.get_sparse_core_info()
    return info.num_cores, info.num_subcores


@functools.lru_cache(maxsize=None)
def _make_sc_gather(n_out, d):
    nc, ns = _sc_info()
    nw = nc * ns
    c = _DMA_ROWS
    assert n_out % (nw * c) == 0
    b_per_w = n_out // nw
    n_chunks = b_per_w // c
    mesh = plsc.VectorSubcoreMesh(core_axis_name="c", subcore_axis_name="s")

    @functools.partial(
        pl.kernel,
        out_type=jax.ShapeDtypeStruct((n_out, d), jnp.float32),
        mesh=mesh,
        scratch_types=[
            pltpu.VMEM((b_per_w,), jnp.int32),
            pltpu.VMEM((16,), jnp.int32),
            pltpu.VMEM((c, d), jnp.float32),
            pltpu.VMEM((c, d), jnp.float32),
            pltpu.VMEM((c, d), jnp.float32),
            pltpu.SemaphoreType.DMA,
            pltpu.SemaphoreType.DMA,
            pltpu.SemaphoreType.DMA,
            pltpu.SemaphoreType.DMA,
            pltpu.SemaphoreType.DMA,
            pltpu.SemaphoreType.DMA,
        ],
    )
    def sc_gather(table_hbm, idx_hbm, thr_hbm, out_hbm, idx_v, thr_v,
                  buf0, buf1, buf2, g_sem0, g_sem1, g_sem2,
                  s_sem0, s_sem1, s_sem2):
        wid = lax.axis_index("s") * nc + lax.axis_index("c")
        base = wid * b_per_w
        pltpu.sync_copy(idx_hbm.at[pl.ds(base, b_per_w)], idx_v)
        pltpu.sync_copy(thr_hbm, thr_v)
        thr = thr_v[...][0]

        bufs = (buf0, buf1, buf2)
        g_sems = (g_sem0, g_sem1, g_sem2)
        s_sems = (s_sem0, s_sem1, s_sem2)

        def active(k):
            return (base + k * c) < thr

        def gather_k(k):
            b = k % 3
            return pltpu.make_async_copy(
                table_hbm.at[idx_v.at[pl.ds(k * c, c)]], bufs[b], g_sems[b])

        def store_k(k):
            b = k % 3
            return pltpu.make_async_copy(
                bufs[b], out_hbm.at[pl.ds(base + k * c, c)], s_sems[b])

        def when_active(j, fn):
            if 0 <= j < n_chunks:
                pl.when(active(j))(lambda: fn(j))

        when_active(0, lambda j: gather_k(j).start())
        when_active(1, lambda j: gather_k(j).start())
        for k in range(n_chunks):
            if k + 2 < n_chunks:
                when_active(k - 1, lambda j: store_k(j).wait())
                when_active(k + 2, lambda j: gather_k(j).start())
            when_active(k, lambda j: gather_k(j).wait())
            when_active(k, lambda j: store_k(j).start())
        for j in range(max(0, n_chunks - 3), n_chunks):
            when_active(j, lambda jj: store_k(jj).wait())

    return sc_gather


@functools.lru_cache(maxsize=None)
def _make_sc_scatter(n_in, d, n_out=None):
    nc, ns = _sc_info()
    nw = nc * ns
    c = _DMA_ROWS
    assert n_in % (nw * c) == 0
    b_per_w = n_in // nw
    n_chunks = b_per_w // c
    mesh = plsc.VectorSubcoreMesh(core_axis_name="c", subcore_axis_name="s")
    out_type = (jax.ShapeDtypeStruct((n_out, d), jnp.float32)
                if n_out is not None else ())

    @functools.partial(
        pl.kernel,
        out_type=out_type,
        mesh=mesh,
        scratch_types=[
            pltpu.VMEM((n_chunks, c), jnp.int32),
            pltpu.VMEM((16,), jnp.int32),
            pltpu.VMEM((c, d), jnp.float32),
            pltpu.VMEM((c, d), jnp.float32),
            pltpu.VMEM((c, d), jnp.float32),
            pltpu.SemaphoreType.DMA,
            pltpu.SemaphoreType.DMA,
            pltpu.SemaphoreType.DMA,
            pltpu.SemaphoreType.DMA,
            pltpu.SemaphoreType.DMA,
            pltpu.SemaphoreType.DMA,
        ],
    )
    def sc_scatter(rows_hbm, idx_hbm, thr_hbm, out_ref, idx_v, thr_v,
                   buf0, buf1, buf2, l_sem0, l_sem1, l_sem2,
                   s_sem0, s_sem1, s_sem2):
        wid = lax.axis_index("s") * nc + lax.axis_index("c")
        base = wid * b_per_w
        pltpu.sync_copy(idx_hbm.at[wid], idx_v)
        pltpu.sync_copy(thr_hbm, thr_v)
        thr = thr_v[...][0]

        bufs = (buf0, buf1, buf2)
        l_sems = (l_sem0, l_sem1, l_sem2)
        s_sems = (s_sem0, s_sem1, s_sem2)

        def active(k):
            return (base + k * c) < thr

        def load_k(k):
            b = k % 3
            return pltpu.make_async_copy(
                rows_hbm.at[pl.ds(base + k * c, c)], bufs[b], l_sems[b])

        def scat_k(k):
            b = k % 3
            return pltpu.make_async_copy(
                bufs[b], out_ref.at[idx_v.at[k]], s_sems[b])

        def when_active(j, fn):
            if 0 <= j < n_chunks:
                pl.when(active(j))(lambda: fn(j))

        when_active(0, lambda j: load_k(j).start())
        when_active(1, lambda j: load_k(j).start())
        for k in range(n_chunks):
            if k + 2 < n_chunks:
                when_active(k - 1, lambda j: scat_k(j).wait())
                when_active(k + 2, lambda j: load_k(j).start())
            when_active(k, lambda j: load_k(j).wait())
            when_active(k, lambda j: scat_k(j).start())
        for j in range(max(0, n_chunks - 3), n_chunks):
            when_active(j, lambda jj: scat_k(jj).wait())

    return sc_scatter


def _grouped_mm_body(g_ref, first_ref, slot_ref, nxt_ref, hn_ref, valid_ref,
                     x_ref, w_hbm, b_ref, o_ref, wbuf, sem):
    i = pl.program_id(0)
    slot = slot_ref[i]

    @pl.when(i == 0)
    def _():
        pltpu.make_async_copy(w_hbm.at[g_ref[0]], wbuf.at[0], sem.at[0]).start()

    @pl.when((first_ref[i] == 1) & (hn_ref[i] == 1))
    def _():
        pltpu.make_async_copy(
            w_hbm.at[nxt_ref[i]], wbuf.at[1 - slot], sem.at[1 - slot]).start()

    @pl.when(first_ref[i] == 1)
    def _():
        pltpu.make_async_copy(
            w_hbm.at[g_ref[i]], wbuf.at[slot], sem.at[slot]).wait()

    @pl.when(valid_ref[i] == 1)
    def _():
        acc = lax.dot_general(
            x_ref[...], wbuf[slot],
            (((1,), (1,)), ((), ())),
            preferred_element_type=jnp.float32,
        )
        o_ref[...] = acc + b_ref[0]


def kernel(input, i_type, weight, bias):
    n_tokens, d_in = input.shape
    n_types, d_out, _ = weight.shape
    m = _M_BLK
    nb = n_tokens // m + n_types
    nbm = nb * m
    nw = _sc_info()[0] * _sc_info()[1]

    tk = jnp.arange(n_types, dtype=jnp.int32)
    counts = jnp.sum(i_type[:, None] == tk[None, :], axis=0, dtype=jnp.int32)
    padded = ((counts + m - 1) // m) * m
    pstart = jnp.concatenate([jnp.zeros((1,), jnp.int32),
                              jnp.cumsum(padded).astype(jnp.int32)])
    start = jnp.concatenate([jnp.zeros((1,), jnp.int32),
                             jnp.cumsum(counts).astype(jnp.int32)])
    t_last = jnp.max(tk * (counts > 0)).astype(jnp.int32)
    total_padded = pstart[n_types]

    order = jnp.argsort(i_type).astype(jnp.int32)

    pos = jnp.arange(nbm, dtype=jnp.int32)
    t_raw = jnp.sum(pstart[1:][None, :] <= pos[:, None], axis=1,
                    dtype=jnp.int32)
    t_eff = jnp.minimum(t_raw, t_last)
    sel = t_eff[:, None] == tk[None, :]
    pstart_t = jnp.sum(jnp.where(sel, pstart[:-1][None, :], 0), axis=1)
    start_t = jnp.sum(jnp.where(sel, start[:-1][None, :], 0), axis=1)
    counts_t = jnp.sum(jnp.where(sel, counts[None, :], 0), axis=1)
    o = (pos - pstart_t) % jnp.maximum(counts_t, 1)
    src = order[jnp.clip(start_t + o, 0, n_tokens - 1)]

    block_g = t_eff[::m]

    bias3 = bias.reshape(n_types, 1, d_out)

    def make_mm(cb, rows):
        grid_spec = pltpu.PrefetchScalarGridSpec(
            num_scalar_prefetch=6,
            grid=(cb,),
            in_specs=[
                pl.BlockSpec((m, d_in), lambda i, *_: (i, 0)),
                pl.BlockSpec(memory_space=pl.ANY),
                pl.BlockSpec((1, 1, d_out), lambda i, g, *_: (g[i], 0, 0)),
            ],
            out_specs=pl.BlockSpec((m, d_out), lambda i, *_: (i, 0)),
            scratch_shapes=[
                pltpu.VMEM((2, d_out, d_in), jnp.float32),
                pltpu.SemaphoreType.DMA((2,)),
            ],
        )
        return pl.pallas_call(
            _grouped_mm_body,
            grid_spec=grid_spec,
            out_shape=jax.ShapeDtypeStruct((rows, d_out), jnp.float32),
        )

    out_ref = None
    b0 = 0
    for ci, cb in enumerate(_CHUNK_BLOCKS):
        rows = cb * m
        src_c = lax.dynamic_slice_in_dim(src, b0 * m, rows)
        g_c = lax.dynamic_slice_in_dim(block_g, b0, cb)

        first = jnp.concatenate([jnp.ones((1,), jnp.int32),
                                 (g_c[1:] != g_c[:-1]).astype(jnp.int32)])
        run = jnp.cumsum(first).astype(jnp.int32) - 1
        slot = run % 2
        idxv = jnp.arange(cb, dtype=jnp.int32)
        cand = jnp.where(first == 1, idxv, cb)
        revmin = lax.cummin(cand[::-1])[::-1]
        nstart = jnp.concatenate([revmin[1:], jnp.full((1,), cb, jnp.int32)])
        hn = (nstart < cb).astype(jnp.int32)
        nxt = g_c[jnp.clip(nstart, 0, cb - 1)]
        thr_c = jnp.clip(total_padded - b0 * m, 0, rows).astype(jnp.int32)
        thr16 = jnp.broadcast_to(thr_c, (16,))
        valid = ((b0 + idxv) * m < total_padded).astype(jnp.int32)

        xs = _make_sc_gather(rows, d_in)(input, src_c, thr16)
        ys = make_mm(cb, rows)(g_c, first, slot, nxt, hn, valid,
                               xs, weight, bias3)
        idx3 = src_c.reshape(nw, rows // (nw * _DMA_ROWS), _DMA_ROWS)
        if ci == 0:
            out_ref = jax.new_ref(
                _make_sc_scatter(rows, d_out, n_tokens)(ys, idx3, thr16))
        else:
            _make_sc_scatter(rows, d_out)(ys, idx3, thr16, out_ref)
        b0 += cb
    return out_ref[...]


# speedup vs baseline: 2.8015x; 2.8015x over previous
import functools

import jax
import jax.numpy as jnp
from jax import lax
from jax.experimental import pallas as pl
from jax.experimental.pallas import tpu as pltpu
from jax.experimental.pallas import tpu_sc as plsc

_M_BLK = 256
_CHUNK_BLOCKS = (20, 20)
_DMA_ROWS = 16


@functools.lru_cache(maxsize=None)
def _sc_info():
    info = plsc.get_sparse_core_info()
    return info.num_cores, info.num_subcores


@functools.lru_cache(maxsize=None)
def _make_sc_gather(n_out, d):
    nc, ns = _sc_info()
    nw = nc * ns
    c = _DMA_ROWS
    assert n_out % (nw * c) == 0
    b_per_w = n_out // nw
    n_chunks = b_per_w // c
    mesh = plsc.VectorSubcoreMesh(core_axis_name="c", subcore_axis_name="s")

    @functools.partial(
        pl.kernel,
        out_type=jax.ShapeDtypeStruct((n_out, d), jnp.float32),
        mesh=mesh,
        scratch_types=[
            pltpu.VMEM((b_per_w,), jnp.int32),
            pltpu.VMEM((16,), jnp.int32),
            pltpu.VMEM((c, d), jnp.float32),
            pltpu.VMEM((c, d), jnp.float32),
            pltpu.SemaphoreType.DMA,
            pltpu.SemaphoreType.DMA,
            pltpu.SemaphoreType.DMA,
            pltpu.SemaphoreType.DMA,
        ],
    )
    def sc_gather(table_hbm, idx_hbm, thr_hbm, out_hbm, idx_v, thr_v,
                  buf0, buf1, g_sem0, g_sem1, s_sem0, s_sem1):
        wid = lax.axis_index("s") * nc + lax.axis_index("c")
        base = wid * b_per_w
        pltpu.sync_copy(idx_hbm.at[pl.ds(base, b_per_w)], idx_v)
        pltpu.sync_copy(thr_hbm, thr_v)
        thr = thr_v[...][0]

        bufs = (buf0, buf1)
        g_sems = (g_sem0, g_sem1)
        s_sems = (s_sem0, s_sem1)

        def active(k):
            return (base + k * c) < thr

        def gather_k(k):
            b = k & 1
            return pltpu.make_async_copy(
                table_hbm.at[idx_v.at[pl.ds(k * c, c)]], bufs[b], g_sems[b])

        def store_k(k):
            b = k & 1
            return pltpu.make_async_copy(
                bufs[b], out_hbm.at[pl.ds(base + k * c, c)], s_sems[b])

        @pl.when(active(0))
        def _():
            gather_k(0).start()

        for k in range(n_chunks):
            if k + 1 < n_chunks:
                if k >= 1:
                    @pl.when(active(k - 1))
                    def _(k=k):
                        store_k(k - 1).wait()

                @pl.when(active(k + 1))
                def _(k=k):
                    gather_k(k + 1).start()

            @pl.when(active(k))
            def _(k=k):
                gather_k(k).wait()
                store_k(k).start()
        if n_chunks >= 2:
            @pl.when(active(n_chunks - 2))
            def _():
                store_k(n_chunks - 2).wait()

        @pl.when(active(n_chunks - 1))
        def _():
            store_k(n_chunks - 1).wait()

    return sc_gather


@functools.lru_cache(maxsize=None)
def _make_sc_scatter(n_in, d, n_out=None):
    nc, ns = _sc_info()
    nw = nc * ns
    c = _DMA_ROWS
    assert n_in % (nw * c) == 0
    b_per_w = n_in // nw
    n_chunks = b_per_w // c
    mesh = plsc.VectorSubcoreMesh(core_axis_name="c", subcore_axis_name="s")
    out_type = (jax.ShapeDtypeStruct((n_out, d), jnp.float32)
                if n_out is not None else ())

    @functools.partial(
        pl.kernel,
        out_type=out_type,
        mesh=mesh,
        scratch_types=[
            pltpu.VMEM((n_chunks, c), jnp.int32),
            pltpu.VMEM((16,), jnp.int32),
            pltpu.VMEM((c, d), jnp.float32),
            pltpu.VMEM((c, d), jnp.float32),
            pltpu.SemaphoreType.DMA,
            pltpu.SemaphoreType.DMA,
            pltpu.SemaphoreType.DMA,
            pltpu.SemaphoreType.DMA,
        ],
    )
    def sc_scatter(rows_hbm, idx_hbm, thr_hbm, out_ref, idx_v, thr_v,
                   buf0, buf1, l_sem0, l_sem1, s_sem0, s_sem1):
        wid = lax.axis_index("s") * nc + lax.axis_index("c")
        base = wid * b_per_w
        pltpu.sync_copy(idx_hbm.at[wid], idx_v)
        pltpu.sync_copy(thr_hbm, thr_v)
        thr = thr_v[...][0]

        bufs = (buf0, buf1)
        l_sems = (l_sem0, l_sem1)
        s_sems = (s_sem0, s_sem1)

        def active(k):
            return (base + k * c) < thr

        def load_k(k):
            b = k & 1
            return pltpu.make_async_copy(
                rows_hbm.at[pl.ds(base + k * c, c)], bufs[b], l_sems[b])

        def scat_k(k):
            b = k & 1
            return pltpu.make_async_copy(
                bufs[b], out_ref.at[idx_v.at[k]], s_sems[b])

        @pl.when(active(0))
        def _():
            load_k(0).start()

        for k in range(n_chunks):
            if k + 1 < n_chunks:
                if k >= 1:
                    @pl.when(active(k - 1))
                    def _(k=k):
                        scat_k(k - 1).wait()

                @pl.when(active(k + 1))
                def _(k=k):
                    load_k(k + 1).start()

            @pl.when(active(k))
            def _(k=k):
                load_k(k).wait()
                scat_k(k).start()
        if n_chunks >= 2:
            @pl.when(active(n_chunks - 2))
            def _():
                scat_k(n_chunks - 2).wait()

        @pl.when(active(n_chunks - 1))
        def _():
            scat_k(n_chunks - 1).wait()

    return sc_scatter


def _grouped_mm_body(g_ref, first_ref, slot_ref, nxt_ref, hn_ref, valid_ref,
                     x_ref, w_hbm, b_ref, o_ref, wbuf, sem):
    i = pl.program_id(0)
    slot = slot_ref[i]

    @pl.when(i == 0)
    def _():
        pltpu.make_async_copy(w_hbm.at[g_ref[0]], wbuf.at[0], sem.at[0]).start()

    @pl.when((first_ref[i] == 1) & (hn_ref[i] == 1))
    def _():
        pltpu.make_async_copy(
            w_hbm.at[nxt_ref[i]], wbuf.at[1 - slot], sem.at[1 - slot]).start()

    @pl.when(first_ref[i] == 1)
    def _():
        pltpu.make_async_copy(
            w_hbm.at[g_ref[i]], wbuf.at[slot], sem.at[slot]).wait()

    @pl.when(valid_ref[i] == 1)
    def _():
        acc = lax.dot_general(
            x_ref[...], wbuf[slot],
            (((1,), (1,)), ((), ())),
            preferred_element_type=jnp.float32,
        )
        o_ref[...] = acc + b_ref[0]


def kernel(input, i_type, weight, bias):
    n_tokens, d_in = input.shape
    n_types, d_out, _ = weight.shape
    m = _M_BLK
    nb = n_tokens // m + n_types
    nbm = nb * m
    nw = _sc_info()[0] * _sc_info()[1]

    tk = jnp.arange(n_types, dtype=jnp.int32)
    counts = jnp.sum(i_type[:, None] == tk[None, :], axis=0, dtype=jnp.int32)
    padded = ((counts + m - 1) // m) * m
    pstart = jnp.concatenate([jnp.zeros((1,), jnp.int32),
                              jnp.cumsum(padded).astype(jnp.int32)])
    start = jnp.concatenate([jnp.zeros((1,), jnp.int32),
                             jnp.cumsum(counts).astype(jnp.int32)])
    t_last = jnp.max(tk * (counts > 0)).astype(jnp.int32)
    total_padded = pstart[n_types]

    order = jnp.argsort(i_type).astype(jnp.int32)

    pos = jnp.arange(nbm, dtype=jnp.int32)
    t_raw = jnp.sum(pstart[1:][None, :] <= pos[:, None], axis=1,
                    dtype=jnp.int32)
    t_eff = jnp.minimum(t_raw, t_last)
    sel = t_eff[:, None] == tk[None, :]
    pstart_t = jnp.sum(jnp.where(sel, pstart[:-1][None, :], 0), axis=1)
    start_t = jnp.sum(jnp.where(sel, start[:-1][None, :], 0), axis=1)
    counts_t = jnp.sum(jnp.where(sel, counts[None, :], 0), axis=1)
    o = (pos - pstart_t) % jnp.maximum(counts_t, 1)
    src = order[jnp.clip(start_t + o, 0, n_tokens - 1)]

    block_g = t_eff[::m]

    bias3 = bias.reshape(n_types, 1, d_out)

    def make_mm(cb, rows):
        grid_spec = pltpu.PrefetchScalarGridSpec(
            num_scalar_prefetch=6,
            grid=(cb,),
            in_specs=[
                pl.BlockSpec((m, d_in), lambda i, *_: (i, 0)),
                pl.BlockSpec(memory_space=pl.ANY),
                pl.BlockSpec((1, 1, d_out), lambda i, g, *_: (g[i], 0, 0)),
            ],
            out_specs=pl.BlockSpec((m, d_out), lambda i, *_: (i, 0)),
            scratch_shapes=[
                pltpu.VMEM((2, d_out, d_in), jnp.float32),
                pltpu.SemaphoreType.DMA((2,)),
            ],
        )
        return pl.pallas_call(
            _grouped_mm_body,
            grid_spec=grid_spec,
            out_shape=jax.ShapeDtypeStruct((rows, d_out), jnp.float32),
        )

    out_ref = None
    b0 = 0
    for ci, cb in enumerate(_CHUNK_BLOCKS):
        rows = cb * m
        src_c = lax.dynamic_slice_in_dim(src, b0 * m, rows)
        g_c = lax.dynamic_slice_in_dim(block_g, b0, cb)

        first = jnp.concatenate([jnp.ones((1,), jnp.int32),
                                 (g_c[1:] != g_c[:-1]).astype(jnp.int32)])
        run = jnp.cumsum(first).astype(jnp.int32) - 1
        slot = run % 2
        idxv = jnp.arange(cb, dtype=jnp.int32)
        cand = jnp.where(first == 1, idxv, cb)
        revmin = lax.cummin(cand[::-1])[::-1]
        nstart = jnp.concatenate([revmin[1:], jnp.full((1,), cb, jnp.int32)])
        hn = (nstart < cb).astype(jnp.int32)
        nxt = g_c[jnp.clip(nstart, 0, cb - 1)]
        thr_c = jnp.clip(total_padded - b0 * m, 0, rows).astype(jnp.int32)
        thr16 = jnp.broadcast_to(thr_c, (16,))
        valid = ((b0 + idxv) * m < total_padded).astype(jnp.int32)

        xs = _make_sc_gather(rows, d_in)(input, src_c, thr16)
        ys = make_mm(cb, rows)(g_c, first, slot, nxt, hn, valid,
                               xs, weight, bias3)
        idx3 = src_c.reshape(nw, rows // (nw * _DMA_ROWS), _DMA_ROWS)
        if ci == 0:
            out_ref = jax.new_ref(
                _make_sc_scatter(rows, d_out, n_tokens)(ys, idx3, thr16))
        else:
            _make_sc_scatter(rows, d_out)(ys, idx3, thr16, out_ref)
        b0 += cb
    return out_ref[...]
